# baseline (speedup 1.0000x reference)
import jax
import jax.numpy as jnp
from jax.experimental import pallas as pl

AN2 = 25
C = 64
H = 64
WW = 64
PS = 4
KNBR = 6
CAND = 9
PN = H // PS
D = C * PS * PS
B = PN * PN
P = CAND * AN2
PIX = H * WW
BIG = 3.0e38


def _patchify_t(x):
    x = x.reshape(1, AN2, C, PN, PS, PN, PS)
    x = jnp.transpose(x, (0, 3, 5, 1, 2, 4, 6))
    return x.reshape(B, AN2, D)


def _patchify(x):
    x = x.reshape(1, AN2, C, PN, PS, PN, PS)
    x = jnp.transpose(x, (0, 3, 5, 2, 4, 6, 1))
    return x.reshape(B, D, AN2)


def _match_kernel(p_ref, *refs):
    cand_refs = refs[:CAND]
    sel_ref = refs[CAND]
    Pv = p_ref[0]
    Ct = jnp.concatenate([r[0] for r in cand_refs], axis=0)
    mm = jax.lax.dot_general(Ct, Pv, (((1,), (0,)), ((), ())),
                             preferred_element_type=jnp.float32)
    qn = jnp.sum(Pv * Pv, axis=0, keepdims=True)
    cn = jnp.sum(Ct * Ct, axis=1, keepdims=True)
    dist = (-2.0 * mm + qn) + cn

    iota = jax.lax.broadcasted_iota(jnp.int32, (P, AN2), 0)
    oh = []
    d = dist
    for _ in range(KNBR):
        m = jnp.min(d, axis=0, keepdims=True)
        am = jnp.min(jnp.where(d == m, iota, P), axis=0, keepdims=True)
        hit = iota == am
        oh.append(hit.astype(jnp.float32))
        d = jnp.where(hit, BIG, d)
    OH = jnp.concatenate(oh, axis=1)
    sel_ref[0] = jax.lax.dot_general(
        Ct, OH, (((0,), (0,)), ((), ())),
        preferred_element_type=jnp.float32)


def _conv_kernel(lf_ref, sel_ref, w1_ref, w2_ref, o_ref):
    lfv = lf_ref[0]
    selv = sel_ref[0]
    A = jnp.dot(w1_ref[...], selv, preferred_element_type=jnp.float32)
    A = jnp.where(A >= 0, A, 0.1 * A)
    X = jnp.concatenate([lfv, A], axis=0)

    pos = jax.lax.broadcasted_iota(jnp.int32, (1, PIX), 1)
    xcol = jax.lax.rem(pos, WW)
    mask_r = (xcol != WW - 1).astype(jnp.float32)
    mask_l = (xcol != 0).astype(jnp.float32)

    acc = jnp.zeros((C, PIX), jnp.float32)
    for t in range(9):
        dy, dx = t // 3 - 1, t % 3 - 1
        s = dy * WW + dx
        Yt = jnp.dot(w2_ref[t], X, preferred_element_type=jnp.float32)
        if s > 0:
            Yt = jnp.concatenate(
                [Yt[:, s:], jnp.zeros((C, s), jnp.float32)], axis=1)
        elif s < 0:
            Yt = jnp.concatenate(
                [jnp.zeros((C, -s), jnp.float32), Yt[:, :PIX + s]], axis=1)
        if dx == 1:
            Yt = Yt * mask_r
        elif dx == -1:
            Yt = Yt * mask_l
        acc = acc + Yt
    o_ref[0] = jnp.where(acc >= 0, acc, 0.1 * acc)


def kernel(lf_fea, W1, W2):
    patch = _patchify(lf_fea)
    p0 = _patchify_t(lf_fea)
    px = _patchify_t(jnp.roll(lf_fea, 2, axis=2))
    py = _patchify_t(jnp.roll(lf_fea, 2, axis=3))
    pxy = _patchify_t(jnp.roll(lf_fea, (2, 2), axis=(2, 3)))

    srcs = []
    for ix in range(3):
        for iy in range(3):
            arr = (p0, py, px, pxy)[(ix % 2) * 2 + (iy % 2)]
            dpy = -(ix // 2)
            dpx = -(iy // 2)

            def imap(b, dpy=dpy, dpx=dpx):
                pyi = (b // PN + dpy) % PN
                pxi = (b % PN + dpx) % PN
                return (pyi * PN + pxi, 0, 0)

            srcs.append((arr, imap))

    sel = pl.pallas_call(
        _match_kernel,
        grid=(B,),
        in_specs=[pl.BlockSpec((1, D, AN2), lambda b: (b, 0, 0))] +
                 [pl.BlockSpec((1, AN2, D), im) for (_, im) in srcs],
        out_specs=pl.BlockSpec((1, D, KNBR * AN2), lambda b: (b, 0, 0)),
        out_shape=jax.ShapeDtypeStruct((B, D, KNBR * AN2), jnp.float32),
    )(patch, *[a for (a, _) in srcs])

    sel = sel.reshape(PN, PN, C, PS, PS, KNBR, AN2)
    sel_img = jnp.transpose(sel, (6, 5, 2, 0, 3, 1, 4)).reshape(AN2, KNBR * C, PIX)

    lfr = lf_fea.reshape(AN2, C, PIX)
    W1r = W1.reshape(C, KNBR * C)
    W2r = jnp.transpose(W2, (2, 3, 0, 1)).reshape(9, C, 2 * C)

    out = pl.pallas_call(
        _conv_kernel,
        grid=(AN2,),
        in_specs=[pl.BlockSpec((1, C, PIX), lambda v: (v, 0, 0)),
                  pl.BlockSpec((1, KNBR * C, PIX), lambda v: (v, 0, 0)),
                  pl.BlockSpec((C, KNBR * C), lambda v: (0, 0)),
                  pl.BlockSpec((9, C, 2 * C), lambda v: (0, 0, 0))],
        out_specs=pl.BlockSpec((1, C, PIX), lambda v: (v, 0, 0)),
        out_shape=jax.ShapeDtypeStruct((AN2, C, PIX), jnp.float32),
    )(lfr, sel_img, W1r, W2r)
    return out.reshape(AN2, C, H, WW)


# speedup vs baseline: 2.5163x; 2.5163x over previous
import jax
import jax.numpy as jnp
from jax.experimental import pallas as pl

AN2 = 25
AN2P = 32
C = 64
H = 64
WW = 64
PS = 4
KNBR = 6
CAND = 9
PN = H // PS
T = PS * PS
D = C * T
B = PN * PN
P = CAND * AN2
PIX = H * WW
BIG = 3.0e38


def _patchify_q(x):
    x = x.reshape(1, AN2, C, PN, PS, PN, PS)
    x = jnp.transpose(x, (0, 3, 5, 4, 6, 2, 1))
    return x.reshape(B, D, AN2)


def _patchify_c(x):
    x = x.reshape(1, AN2, C, PN, PS, PN, PS)
    x = jnp.transpose(x, (0, 3, 5, 1, 4, 6, 2))
    return x.reshape(B, AN2, D)


def _match_kernel(p_ref, *refs):
    cand_refs = refs[:CAND]
    w1_ref = refs[CAND]
    a_ref = refs[CAND + 1]
    Pv = p_ref[0]
    Ct = jnp.concatenate([r[0] for r in cand_refs], axis=0)
    mm = jax.lax.dot_general(Ct, Pv, (((1,), (0,)), ((), ())),
                             preferred_element_type=jnp.float32)
    qn = jnp.sum(Pv * Pv, axis=0, keepdims=True)
    cn = jnp.sum(Ct * Ct, axis=1, keepdims=True)
    dist = (-2.0 * mm + qn) + cn

    iota = jax.lax.broadcasted_iota(jnp.int32, (P, AN2), 0)
    oh = []
    d = dist
    for _ in range(KNBR):
        m = jnp.min(d, axis=0, keepdims=True)
        am = jnp.min(jnp.where(d == m, iota, P), axis=0, keepdims=True)
        hit = iota == am
        oh.append(hit.astype(jnp.float32))
        oh.append(jnp.zeros((P, AN2P - AN2), jnp.float32))
        d = jnp.where(hit, BIG, d)
    OH = jnp.concatenate(oh, axis=1)
    sel = jax.lax.dot_general(
        Ct, OH, (((0,), (0,)), ((), ())),
        preferred_element_type=jnp.float32)
    M = jnp.concatenate(
        [jnp.concatenate([sel[t * C:(t + 1) * C, k * AN2P:(k + 1) * AN2P]
                          for t in range(T)], axis=1)
         for k in range(KNBR)],
        axis=0)
    A = jnp.dot(w1_ref[...], M, preferred_element_type=jnp.float32)
    a_ref[0] = jnp.where(A >= 0, A, 0.1 * A)


def _conv_kernel(lf_ref, a_ref, w2_ref, o_ref):
    lfv = lf_ref[0]
    X = jnp.concatenate([lfv, a_ref[0]], axis=0)

    pos = jax.lax.broadcasted_iota(jnp.int32, (1, PIX), 1)
    xcol = jax.lax.rem(pos, WW)
    mask_r = (xcol != WW - 1).astype(jnp.float32)
    mask_l = (xcol != 0).astype(jnp.float32)

    acc = jnp.zeros((C, PIX), jnp.float32)
    for t in range(9):
        dy, dx = t // 3 - 1, t % 3 - 1
        s = dy * WW + dx
        Yt = jnp.dot(w2_ref[t], X, preferred_element_type=jnp.float32)
        if s > 0:
            Yt = jnp.concatenate(
                [Yt[:, s:], jnp.zeros((C, s), jnp.float32)], axis=1)
        elif s < 0:
            Yt = jnp.concatenate(
                [jnp.zeros((C, -s), jnp.float32), Yt[:, :PIX + s]], axis=1)
        if dx == 1:
            Yt = Yt * mask_r
        elif dx == -1:
            Yt = Yt * mask_l
        acc = acc + Yt
    o_ref[0] = jnp.where(acc >= 0, acc, 0.1 * acc)


def kernel(lf_fea, W1, W2):
    patch = _patchify_q(lf_fea)
    p0 = _patchify_c(lf_fea)
    px = _patchify_c(jnp.roll(lf_fea, 2, axis=2))
    py = _patchify_c(jnp.roll(lf_fea, 2, axis=3))
    pxy = _patchify_c(jnp.roll(lf_fea, (2, 2), axis=(2, 3)))

    srcs = []
    for ix in range(3):
        for iy in range(3):
            arr = (p0, py, px, pxy)[(ix % 2) * 2 + (iy % 2)]
            dpy = -(ix // 2)
            dpx = -(iy // 2)

            def imap(b, dpy=dpy, dpx=dpx):
                pyi = (b // PN + dpy) % PN
                pxi = (b % PN + dpx) % PN
                return (pyi * PN + pxi, 0, 0)

            srcs.append((arr, imap))

    W1r = W1.reshape(C, KNBR * C)
    A = pl.pallas_call(
        _match_kernel,
        grid=(B,),
        in_specs=[pl.BlockSpec((1, D, AN2), lambda b: (b, 0, 0))] +
                 [pl.BlockSpec((1, AN2, D), im) for (_, im) in srcs] +
                 [pl.BlockSpec((C, KNBR * C), lambda b: (0, 0))],
        out_specs=pl.BlockSpec((1, C, T * AN2P), lambda b: (b, 0, 0)),
        out_shape=jax.ShapeDtypeStruct((B, C, T * AN2P), jnp.float32),
    )(patch, *[a for (a, _) in srcs], W1r)

    A = A.reshape(PN, PN, C, PS, PS, AN2P)[..., :AN2]
    Aimg = jnp.transpose(A, (5, 2, 0, 3, 1, 4)).reshape(AN2, C, PIX)

    lfr = lf_fea.reshape(AN2, C, PIX)
    W2r = jnp.transpose(W2, (2, 3, 0, 1)).reshape(9, C, 2 * C)

    out = pl.pallas_call(
        _conv_kernel,
        grid=(AN2,),
        in_specs=[pl.BlockSpec((1, C, PIX), lambda v: (v, 0, 0)),
                  pl.BlockSpec((1, C, PIX), lambda v: (v, 0, 0)),
                  pl.BlockSpec((9, C, 2 * C), lambda v: (0, 0, 0))],
        out_specs=pl.BlockSpec((1, C, PIX), lambda v: (v, 0, 0)),
        out_shape=jax.ShapeDtypeStruct((AN2, C, PIX), jnp.float32),
    )(lfr, Aimg, W2r)
    return out.reshape(AN2, C, H, WW)
